# baseline (speedup 1.0000x reference)
import jax
import jax.numpy as jnp
from jax import lax
from jax.experimental import pallas as pl
from jax.experimental.pallas import tpu as pltpu
from jax.experimental.pallas import tpu_sc as plsc

NIMG = 8
NCH = 2
H = 512
W = 512
NPIX = H * W
NLBL = 16
NC = 2
NS = 16
NW = NC * NS
ROWS_PW = H // NW
CHUNKS = ROWS_PW * W // 16
WCH = W // 16
REC = NIMG * 3 * 16
NBANK = 8

F32 = jnp.float32
I32 = jnp.int32


def _rsqrt(s, iters=2):
    i = plsc.bitcast(s, I32)
    i = jnp.int32(0x5F3759DF) - lax.shift_right_logical(i, 1)
    y = plsc.bitcast(i, F32)
    hs = jnp.float32(0.5) * s
    for _ in range(iters):
        y = y * (jnp.float32(1.5) - hs * y * y)
    return y


def _zeros16():
    return jnp.zeros((16,), F32)


def _rc(i):
    return lax.bitwise_and(i, jnp.int32(0xF)), lax.bitwise_and(i, jnp.int32(0x1F0))


def _phase_a(x_hbm, t_hbm, part_hbm, bufs, tabs, out_v, sems):
    wid = lax.axis_index("s") * NC + lax.axis_index("c")
    row0 = wid * ROWS_PW
    lanes = lax.iota(I32, 16)
    lanes16 = lanes * 16
    ones = jnp.ones((16,), F32)
    cnt_t, s0_t, s1_t = tabs

    @plsc.parallel_loop(0, NIMG * NBANK * NLBL, 1, unroll=8)
    def _(r):
        cnt_t[pl.ds(r * 16, 16)] = _zeros16()
        s0_t[pl.ds(r * 16, 16)] = _zeros16()
        s1_t[pl.ds(r * 16, 16)] = _zeros16()

    def start(img, b):
        x0_v, x1_v, t_v = bufs[b]
        return (
            pltpu.async_copy(
                x_hbm.at[pl.ds((img * NCH + 0) * H + row0, ROWS_PW), :], x0_v, sems[b]
            ),
            pltpu.async_copy(
                x_hbm.at[pl.ds((img * NCH + 1) * H + row0, ROWS_PW), :], x1_v, sems[b]
            ),
            pltpu.async_copy(t_hbm.at[pl.ds(img * H + row0, ROWS_PW), :], t_v, sems[b]),
        )

    pending = start(0, 0)
    for img in range(NIMG):
        b = img % 2
        x0_v, x1_v, t_v = bufs[b]
        for cp in pending:
            cp.wait()
        if img + 1 < NIMG:
            pending = start(img + 1, 1 - b)

        img_base = img * NBANK * NLBL * 16

        @plsc.parallel_loop(0, CHUNKS, 1, unroll=NBANK)
        def _(i):
            r, c = _rc(i)
            col = jnp.full((16,), c, I32) + lanes
            row = jnp.full((16,), r, I32)
            bank = lax.bitwise_and(i, jnp.int32(NBANK - 1)) * (NLBL * 16) + jnp.int32(img_base)
            t16 = plsc.load_gather(t_v, [row, col])
            x0 = plsc.load_gather(x0_v, [row, col])
            x1 = plsc.load_gather(x1_v, [row, col])
            slot = jnp.full((16,), bank, I32) + lanes16 + t16
            plsc.addupdate_scatter(cnt_t, [slot], ones)
            plsc.addupdate_scatter(s0_t, [slot], x0)
            plsc.addupdate_scatter(s1_t, [slot], x1)

    for img in range(NIMG):
        img_base = img * NBANK * NLBL * 16
        for comp, tab in enumerate((cnt_t, s0_t, s1_t)):

            @plsc.parallel_loop(0, NBANK * 16, 1, unroll=8, carry=_zeros16())
            def acc_tab(j, acc, tab=tab, img_base=img_base):
                return acc + tab[pl.ds(img_base + j * 16, 16)]

            out_v[pl.ds((img * 3 + comp) * 16, 16)] = acc_tab

    pltpu.sync_copy(out_v, part_hbm.at[pl.ds(wid * REC, REC)])


def _phase_b(x_hbm, t_hbm, part_hbm, pairs_hbm, out_hbm, part_v, bufs, tabs, pair_v, stage_v, sems):
    wid = lax.axis_index("s") * NC + lax.axis_index("c")
    row0 = wid * ROWS_PW
    lanes = lax.iota(I32, 16)
    m0_t, m1_t, cf_t, vd_t = tabs

    def start(img, b):
        x0_v, x1_v, t_v = bufs[b]
        return (
            pltpu.async_copy(
                x_hbm.at[pl.ds((img * NCH + 0) * H + row0, ROWS_PW), :], x0_v, sems[b]
            ),
            pltpu.async_copy(
                x_hbm.at[pl.ds((img * NCH + 1) * H + row0, ROWS_PW), :], x1_v, sems[b]
            ),
            pltpu.async_copy(t_hbm.at[pl.ds(img * H + row0, ROWS_PW), :], t_v, sems[b]),
        )

    pending = start(0, 0)
    pltpu.sync_copy(part_hbm, part_v)
    pltpu.sync_copy(pairs_hbm, pair_v)

    lbl_ok = jnp.where((lanes >= 1) & (lanes <= 8), jnp.float32(1.0), jnp.float32(0.0))

    for img in range(NIMG):

        @plsc.parallel_loop(0, NW, 1, unroll=4, carry=(_zeros16(), _zeros16(), _zeros16()))
        def sums(w, carry, img=img):
            cnt, s0, s1 = carry
            rec = w * REC + img * 48
            cnt = cnt + part_v[pl.ds(rec, 16)]
            s0 = s0 + part_v[pl.ds(rec + 16, 16)]
            s1 = s1 + part_v[pl.ds(rec + 32, 16)]
            return cnt, s0, s1

        cnt, s0, s1 = sums
        safe = jnp.maximum(cnt, jnp.float32(1.0))
        vd = jnp.where(cnt > jnp.float32(1.0), jnp.float32(1.0), jnp.float32(0.0)) * lbl_ok
        m0_t[pl.ds(img * 16, 16)] = s0 / safe
        m1_t[pl.ds(img * 16, 16)] = s1 / safe
        cf_t[pl.ds(img * 16, 16)] = vd / safe
        vd_t[pl.ds(img * 16, 16)] = vd

    acc_total = _zeros16()
    for img in range(NIMG):
        b = img % 2
        x0_v, x1_v, t_v = bufs[b]
        for cp in pending:
            cp.wait()
        if img + 1 < NIMG:
            pending = start(img + 1, 1 - b)
        m0_img = m0_t.at[pl.ds(img * 16, 16)]
        m1_img = m1_t.at[pl.ds(img * 16, 16)]
        cf_img = cf_t.at[pl.ds(img * 16, 16)]

        @plsc.parallel_loop(0, CHUNKS, 1, unroll=6, carry=_zeros16())
        def acc_img(i, acc):
            r, c = _rc(i)
            col = jnp.full((16,), c, I32) + lanes
            row = jnp.full((16,), r, I32)
            t16 = plsc.load_gather(t_v, [row, col])
            x0 = plsc.load_gather(x0_v, [row, col])
            x1 = plsc.load_gather(x1_v, [row, col])
            m0 = plsc.load_gather(m0_img, [t16])
            m1 = plsc.load_gather(m1_img, [t16])
            cf = plsc.load_gather(cf_img, [t16])
            dx = x0 - m0
            dy = x1 - m1
            s = dx * dx + dy * dy + jnp.float32(1e-8)
            d = s * _rsqrt(s)
            u = jnp.maximum(d - jnp.float32(0.5), jnp.float32(0.0))
            return acc + u * u * cf

        acc_total = acc_total + acc_img

    img_off = jnp.full((16,), lax.rem(wid, NIMG) * 16, I32)
    accp = _zeros16()
    for half in range(2):
        pi = pair_v[pl.ds(half * 16, 16)]
        pj = pair_v[pl.ds(32 + half * 16, 16)]
        mi0 = plsc.load_gather(m0_t, [img_off + pi])
        mj0 = plsc.load_gather(m0_t, [img_off + pj])
        mi1 = plsc.load_gather(m1_t, [img_off + pi])
        mj1 = plsc.load_gather(m1_t, [img_off + pj])
        vi = plsc.load_gather(vd_t, [img_off + pi])
        vj = plsc.load_gather(vd_t, [img_off + pj])
        dmx = mi0 - mj0
        dmy = mi1 - mj1
        sp = dmx * dmx + dmy * dmy + jnp.float32(1e-8)
        dp = sp * _rsqrt(sp)
        up = jnp.maximum(jnp.float32(1.0) - dp, jnp.float32(0.0))
        pm = vi * vj * jnp.where(pi < pj, jnp.float32(1.0), jnp.float32(0.0))
        accp = accp + up * up * pm

    vdrow = plsc.load_gather(vd_t, [img_off + lanes])
    n_v = jnp.full((16,), jnp.sum(vdrow), F32)
    ld_v = jnp.full((16,), jnp.sum(accp), F32)
    dist_v = ld_v / (n_v - jnp.float32(1.0) + jnp.float32(1e-8))
    wid_v = jnp.full((16,), wid, I32)
    take = (lanes == 0) & (n_v > jnp.float32(1.0)) & (wid_v < NIMG)
    dist_row = jnp.where(take, dist_v, _zeros16())

    row = (acc_total + dist_row) * jnp.float32(1.0 / NIMG)
    stage_v[pl.ds(0, 16)] = row
    pltpu.sync_copy(stage_v, out_hbm.at[pl.ds(wid * 16, 16)])


def _pair_table():
    pi = []
    pj = []
    for i in range(1, 9):
        for j in range(i + 1, 9):
            pi.append(i)
            pj.append(j)
    while len(pi) < 32:
        pi.append(0)
        pj.append(0)
    return jnp.asarray(pi + pj, dtype=jnp.int32)


def _pix_bufs():
    return [
        (
            pltpu.VMEM((ROWS_PW, W), F32),
            pltpu.VMEM((ROWS_PW, W), F32),
            pltpu.VMEM((ROWS_PW, W), I32),
        )
        for _ in range(2)
    ]


@jax.jit
def kernel(inputs, targets):
    x = inputs.reshape(NIMG * NCH * H, W)
    t = targets.reshape(NIMG * H, W).astype(jnp.int32)
    pairs = _pair_table()

    mesh = plsc.VectorSubcoreMesh(core_axis_name="c", subcore_axis_name="s")
    params = pltpu.CompilerParams(needs_layout_passes=False, use_tc_tiling_on_sc=True)

    phase_a = pl.kernel(
        _phase_a,
        out_type=jax.ShapeDtypeStruct((NW * REC,), F32),
        mesh=mesh,
        scratch_types=[
            _pix_bufs(),
            [pltpu.VMEM((NIMG * NBANK * NLBL * 16,), F32) for _ in range(3)],
            pltpu.VMEM((REC,), F32),
            [pltpu.SemaphoreType.DMA for _ in range(2)],
        ],
        compiler_params=params,
        name="instance_loss_segsums",
    )
    part = phase_a(x, t)

    phase_b = pl.kernel(
        _phase_b,
        out_type=jax.ShapeDtypeStruct((NW * 16,), F32),
        mesh=mesh,
        scratch_types=[
            pltpu.VMEM((NW * REC,), F32),
            _pix_bufs(),
            [pltpu.VMEM((NIMG * 16,), F32) for _ in range(4)],
            pltpu.VMEM((64,), I32),
            pltpu.VMEM((16,), F32),
            [pltpu.SemaphoreType.DMA for _ in range(2)],
        ],
        compiler_params=params,
        name="instance_loss_var_dist",
    )
    out = phase_b(x, t, part, pairs)
    return jnp.sum(out).reshape(1)


# speedup vs baseline: 1.1117x; 1.1117x over previous
import jax
import jax.numpy as jnp
from jax import lax
from jax.experimental import pallas as pl
from jax.experimental.pallas import tpu as pltpu
from jax.experimental.pallas import tpu_sc as plsc

NIMG = 8
NCH = 2
H = 512
W = 512
NPIX = H * W
NLBL = 16
NC = 2
NS = 16
NW = NC * NS
ROWS_PW = H // NW
CHUNKS = ROWS_PW * W // 16
WCH = W // 16
REC = NIMG * 3 * 16
NBANK = 4

F32 = jnp.float32
I32 = jnp.int32


def _rsqrt(s, iters=2):
    i = plsc.bitcast(s, I32)
    i = jnp.int32(0x5F3759DF) - lax.shift_right_logical(i, 1)
    y = plsc.bitcast(i, F32)
    hs = jnp.float32(0.5) * s
    for _ in range(iters):
        y = y * (jnp.float32(1.5) - hs * y * y)
    return y


def _zeros16():
    return jnp.zeros((16,), F32)


def _rc(i):
    return lax.bitwise_and(i, jnp.int32(0xF)), lax.bitwise_and(i, jnp.int32(0x1F0))


def _phase_a(x_hbm, t_hbm, part_hbm, bufs, tabs, out_v, sems):
    wid = lax.axis_index("s") * NC + lax.axis_index("c")
    row0 = wid * ROWS_PW
    lanes = lax.iota(I32, 16)
    lanes16 = lanes * 16
    ones = jnp.ones((16,), F32)
    cnt_t, s0_t, s1_t = tabs

    @plsc.parallel_loop(0, NIMG * NBANK * NLBL, 1, unroll=8)
    def _(r):
        cnt_t[pl.ds(r * 16, 16)] = _zeros16()
        s0_t[pl.ds(r * 16, 16)] = _zeros16()
        s1_t[pl.ds(r * 16, 16)] = _zeros16()

    def start(img, b):
        x0_v, x1_v, t_v = bufs[b]
        return (
            pltpu.async_copy(
                x_hbm.at[pl.ds((img * NCH + 0) * H + row0, ROWS_PW), :], x0_v, sems[b]
            ),
            pltpu.async_copy(
                x_hbm.at[pl.ds((img * NCH + 1) * H + row0, ROWS_PW), :], x1_v, sems[b]
            ),
            pltpu.async_copy(t_hbm.at[pl.ds(img * H + row0, ROWS_PW), :], t_v, sems[b]),
        )

    pending = start(0, 0)
    for img in range(NIMG):
        b = img % 2
        x0_v, x1_v, t_v = bufs[b]
        for cp in pending:
            cp.wait()
        if img + 1 < NIMG:
            pending = start(img + 1, 1 - b)

        img_base = img * NBANK * NLBL * 16

        @plsc.parallel_loop(0, CHUNKS, 1, unroll=NBANK)
        def _(i):
            r, c = _rc(i)
            col = jnp.full((16,), c, I32) + lanes
            row = jnp.full((16,), r, I32)
            bank = lax.bitwise_and(i, jnp.int32(NBANK - 1)) * (NLBL * 16) + jnp.int32(img_base)
            t16 = plsc.load_gather(t_v, [row, col])
            x0 = plsc.load_gather(x0_v, [row, col])
            x1 = plsc.load_gather(x1_v, [row, col])
            slot = jnp.full((16,), bank, I32) + lanes16 + t16
            plsc.addupdate_scatter(cnt_t, [slot], ones)
            plsc.addupdate_scatter(s0_t, [slot], x0)
            plsc.addupdate_scatter(s1_t, [slot], x1)

    for img in range(NIMG):
        img_base = img * NBANK * NLBL * 16
        for comp, tab in enumerate((cnt_t, s0_t, s1_t)):

            @plsc.parallel_loop(0, NBANK * 16, 1, unroll=8, carry=_zeros16())
            def acc_tab(j, acc, tab=tab, img_base=img_base):
                return acc + tab[pl.ds(img_base + j * 16, 16)]

            out_v[pl.ds((img * 3 + comp) * 16, 16)] = acc_tab

    pltpu.sync_copy(out_v, part_hbm.at[pl.ds(wid * REC, REC)])


def _phase_b(x_hbm, t_hbm, part_hbm, pairs_hbm, out_hbm, part_v, bufs, tabs, pair_v, stage_v, sems):
    wid = lax.axis_index("s") * NC + lax.axis_index("c")
    row0 = wid * ROWS_PW
    lanes = lax.iota(I32, 16)
    m0_t, m1_t, cf_t, vd_t = tabs

    def start(img, b):
        x0_v, x1_v, t_v = bufs[b]
        return (
            pltpu.async_copy(
                x_hbm.at[pl.ds((img * NCH + 0) * H + row0, ROWS_PW), :], x0_v, sems[b]
            ),
            pltpu.async_copy(
                x_hbm.at[pl.ds((img * NCH + 1) * H + row0, ROWS_PW), :], x1_v, sems[b]
            ),
            pltpu.async_copy(t_hbm.at[pl.ds(img * H + row0, ROWS_PW), :], t_v, sems[b]),
        )

    pending = start(0, 0)
    pltpu.sync_copy(part_hbm, part_v)
    pltpu.sync_copy(pairs_hbm, pair_v)

    lbl_ok = jnp.where((lanes >= 1) & (lanes <= 8), jnp.float32(1.0), jnp.float32(0.0))

    for img in range(NIMG):

        @plsc.parallel_loop(0, NW, 1, unroll=4, carry=(_zeros16(), _zeros16(), _zeros16()))
        def sums(w, carry, img=img):
            cnt, s0, s1 = carry
            rec = w * REC + img * 48
            cnt = cnt + part_v[pl.ds(rec, 16)]
            s0 = s0 + part_v[pl.ds(rec + 16, 16)]
            s1 = s1 + part_v[pl.ds(rec + 32, 16)]
            return cnt, s0, s1

        cnt, s0, s1 = sums
        safe = jnp.maximum(cnt, jnp.float32(1.0))
        vd = jnp.where(cnt > jnp.float32(1.0), jnp.float32(1.0), jnp.float32(0.0)) * lbl_ok
        m0_t[pl.ds(img * 16, 16)] = s0 / safe
        m1_t[pl.ds(img * 16, 16)] = s1 / safe
        cf_t[pl.ds(img * 16, 16)] = vd / safe
        vd_t[pl.ds(img * 16, 16)] = vd

    acc_total = _zeros16()
    for img in range(NIMG):
        b = img % 2
        x0_v, x1_v, t_v = bufs[b]
        for cp in pending:
            cp.wait()
        if img + 1 < NIMG:
            pending = start(img + 1, 1 - b)
        m0_img = m0_t.at[pl.ds(img * 16, 16)]
        m1_img = m1_t.at[pl.ds(img * 16, 16)]
        cf_img = cf_t.at[pl.ds(img * 16, 16)]

        @plsc.parallel_loop(0, CHUNKS, 1, unroll=4, carry=_zeros16())
        def acc_img(i, acc):
            r, c = _rc(i)
            col = jnp.full((16,), c, I32) + lanes
            row = jnp.full((16,), r, I32)
            t16 = plsc.load_gather(t_v, [row, col])
            x0 = plsc.load_gather(x0_v, [row, col])
            x1 = plsc.load_gather(x1_v, [row, col])
            m0 = plsc.load_gather(m0_img, [t16])
            m1 = plsc.load_gather(m1_img, [t16])
            cf = plsc.load_gather(cf_img, [t16])
            dx = x0 - m0
            dy = x1 - m1
            s = dx * dx + dy * dy + jnp.float32(1e-8)
            d = s * _rsqrt(s)
            u = jnp.maximum(d - jnp.float32(0.5), jnp.float32(0.0))
            return acc + u * u * cf

        acc_total = acc_total + acc_img

    img_off = jnp.full((16,), lax.rem(wid, NIMG) * 16, I32)
    accp = _zeros16()
    for half in range(2):
        pi = pair_v[pl.ds(half * 16, 16)]
        pj = pair_v[pl.ds(32 + half * 16, 16)]
        mi0 = plsc.load_gather(m0_t, [img_off + pi])
        mj0 = plsc.load_gather(m0_t, [img_off + pj])
        mi1 = plsc.load_gather(m1_t, [img_off + pi])
        mj1 = plsc.load_gather(m1_t, [img_off + pj])
        vi = plsc.load_gather(vd_t, [img_off + pi])
        vj = plsc.load_gather(vd_t, [img_off + pj])
        dmx = mi0 - mj0
        dmy = mi1 - mj1
        sp = dmx * dmx + dmy * dmy + jnp.float32(1e-8)
        dp = sp * _rsqrt(sp)
        up = jnp.maximum(jnp.float32(1.0) - dp, jnp.float32(0.0))
        pm = vi * vj * jnp.where(pi < pj, jnp.float32(1.0), jnp.float32(0.0))
        accp = accp + up * up * pm

    vdrow = plsc.load_gather(vd_t, [img_off + lanes])
    n_v = jnp.full((16,), jnp.sum(vdrow), F32)
    ld_v = jnp.full((16,), jnp.sum(accp), F32)
    dist_v = ld_v / (n_v - jnp.float32(1.0) + jnp.float32(1e-8))
    wid_v = jnp.full((16,), wid, I32)
    take = (lanes == 0) & (n_v > jnp.float32(1.0)) & (wid_v < NIMG)
    dist_row = jnp.where(take, dist_v, _zeros16())

    row = (acc_total + dist_row) * jnp.float32(1.0 / NIMG)
    stage_v[pl.ds(0, 16)] = row
    pltpu.sync_copy(stage_v, out_hbm.at[pl.ds(wid * 16, 16)])


def _pair_table():
    pi = []
    pj = []
    for i in range(1, 9):
        for j in range(i + 1, 9):
            pi.append(i)
            pj.append(j)
    while len(pi) < 32:
        pi.append(0)
        pj.append(0)
    return jnp.asarray(pi + pj, dtype=jnp.int32)


def _pix_bufs():
    return [
        (
            pltpu.VMEM((ROWS_PW, W), F32),
            pltpu.VMEM((ROWS_PW, W), F32),
            pltpu.VMEM((ROWS_PW, W), I32),
        )
        for _ in range(2)
    ]


@jax.jit
def kernel(inputs, targets):
    x = inputs.reshape(NIMG * NCH * H, W)
    t = targets.reshape(NIMG * H, W).astype(jnp.int32)
    pairs = _pair_table()

    mesh = plsc.VectorSubcoreMesh(core_axis_name="c", subcore_axis_name="s")
    params = pltpu.CompilerParams(needs_layout_passes=False, use_tc_tiling_on_sc=True)

    phase_a = pl.kernel(
        _phase_a,
        out_type=jax.ShapeDtypeStruct((NW * REC,), F32),
        mesh=mesh,
        scratch_types=[
            _pix_bufs(),
            [pltpu.VMEM((NIMG * NBANK * NLBL * 16,), F32) for _ in range(3)],
            pltpu.VMEM((REC,), F32),
            [pltpu.SemaphoreType.DMA for _ in range(2)],
        ],
        compiler_params=params,
        name="instance_loss_segsums",
    )
    part = phase_a(x, t)

    phase_b = pl.kernel(
        _phase_b,
        out_type=jax.ShapeDtypeStruct((NW * 16,), F32),
        mesh=mesh,
        scratch_types=[
            pltpu.VMEM((NW * REC,), F32),
            _pix_bufs(),
            [pltpu.VMEM((NIMG * 16,), F32) for _ in range(4)],
            pltpu.VMEM((64,), I32),
            pltpu.VMEM((16,), F32),
            [pltpu.SemaphoreType.DMA for _ in range(2)],
        ],
        compiler_params=params,
        name="instance_loss_var_dist",
    )
    out = phase_b(x, t, part, pairs)
    return jnp.sum(out).reshape(1)


# speedup vs baseline: 1.1551x; 1.0390x over previous
import jax
import jax.numpy as jnp
from jax import lax
from jax.experimental import pallas as pl
from jax.experimental.pallas import tpu as pltpu
from jax.experimental.pallas import tpu_sc as plsc

NIMG = 8
NCH = 2
H = 512
W = 512
NPIX = H * W
NLBL = 16
NC = 2
NS = 16
NW = NC * NS
ROWS_PW = H // NW
CHUNKS = ROWS_PW * W // 16
WCH = W // 16
REC = NIMG * 3 * 16
NBANK = 4

F32 = jnp.float32
I32 = jnp.int32


def _rsqrt(s, iters=1):
    i = plsc.bitcast(s, I32)
    i = jnp.int32(0x5F3759DF) - lax.shift_right_logical(i, 1)
    y = plsc.bitcast(i, F32)
    hs = jnp.float32(0.5) * s
    for _ in range(iters):
        y = y * (jnp.float32(1.5) - hs * y * y)
    return y


def _zeros16():
    return jnp.zeros((16,), F32)


def _rc(i):
    return lax.bitwise_and(i, jnp.int32(0xF)), lax.bitwise_and(i, jnp.int32(0x1F0))


def _phase_a(x_hbm, t_hbm, part_hbm, bufs, tabs, out_v, sems):
    wid = lax.axis_index("s") * NC + lax.axis_index("c")
    row0 = wid * ROWS_PW
    lanes = lax.iota(I32, 16)
    lanes16 = lanes * 16
    ones = jnp.ones((16,), F32)
    cnt_t, s0_t, s1_t = tabs

    @plsc.parallel_loop(0, NIMG * NBANK * NLBL, 1, unroll=8)
    def _(r):
        cnt_t[pl.ds(r * 16, 16)] = _zeros16()
        s0_t[pl.ds(r * 16, 16)] = _zeros16()
        s1_t[pl.ds(r * 16, 16)] = _zeros16()

    def start(img, b):
        x0_v, x1_v, t_v = bufs[b]
        return (
            pltpu.async_copy(
                x_hbm.at[pl.ds((img * NCH + 0) * H + row0, ROWS_PW), :], x0_v, sems[b]
            ),
            pltpu.async_copy(
                x_hbm.at[pl.ds((img * NCH + 1) * H + row0, ROWS_PW), :], x1_v, sems[b]
            ),
            pltpu.async_copy(t_hbm.at[pl.ds(img * H + row0, ROWS_PW), :], t_v, sems[b]),
        )

    pending = start(0, 0)
    for img in range(NIMG):
        b = img % 2
        x0_v, x1_v, t_v = bufs[b]
        for cp in pending:
            cp.wait()
        if img + 1 < NIMG:
            pending = start(img + 1, 1 - b)

        img_base = img * NBANK * NLBL * 16

        @plsc.parallel_loop(0, CHUNKS, 1, unroll=NBANK)
        def _(i):
            r, c = _rc(i)
            col = jnp.full((16,), c, I32) + lanes
            row = jnp.full((16,), r, I32)
            bank = lax.bitwise_and(i, jnp.int32(NBANK - 1)) * (NLBL * 16) + jnp.int32(img_base)
            t16 = plsc.load_gather(t_v, [row, col])
            x0 = plsc.load_gather(x0_v, [row, col])
            x1 = plsc.load_gather(x1_v, [row, col])
            slot = jnp.full((16,), bank, I32) + lanes16 + t16
            plsc.addupdate_scatter(cnt_t, [slot], ones)
            plsc.addupdate_scatter(s0_t, [slot], x0)
            plsc.addupdate_scatter(s1_t, [slot], x1)

    for img in range(NIMG):
        img_base = img * NBANK * NLBL * 16
        for comp, tab in enumerate((cnt_t, s0_t, s1_t)):

            @plsc.parallel_loop(0, NBANK * 16, 1, unroll=8, carry=_zeros16())
            def acc_tab(j, acc, tab=tab, img_base=img_base):
                return acc + tab[pl.ds(img_base + j * 16, 16)]

            out_v[pl.ds((img * 3 + comp) * 16, 16)] = acc_tab

    pltpu.sync_copy(out_v, part_hbm.at[pl.ds(wid * REC, REC)])


def _phase_b(x_hbm, t_hbm, part_hbm, pairs_hbm, out_hbm, part_v, bufs, tabs, pair_v, stage_v, sems):
    wid = lax.axis_index("s") * NC + lax.axis_index("c")
    row0 = wid * ROWS_PW
    lanes = lax.iota(I32, 16)
    m0_t, m1_t, cf_t, vd_t = tabs

    def start(img, b):
        x0_v, x1_v, t_v = bufs[b]
        return (
            pltpu.async_copy(
                x_hbm.at[pl.ds((img * NCH + 0) * H + row0, ROWS_PW), :], x0_v, sems[b]
            ),
            pltpu.async_copy(
                x_hbm.at[pl.ds((img * NCH + 1) * H + row0, ROWS_PW), :], x1_v, sems[b]
            ),
            pltpu.async_copy(t_hbm.at[pl.ds(img * H + row0, ROWS_PW), :], t_v, sems[b]),
        )

    pending = start(0, 0)
    pltpu.sync_copy(part_hbm, part_v)
    pltpu.sync_copy(pairs_hbm, pair_v)

    lbl_ok = jnp.where((lanes >= 1) & (lanes <= 8), jnp.float32(1.0), jnp.float32(0.0))

    for img in range(NIMG):

        @plsc.parallel_loop(0, NW, 1, unroll=4, carry=(_zeros16(), _zeros16(), _zeros16()))
        def sums(w, carry, img=img):
            cnt, s0, s1 = carry
            rec = w * REC + img * 48
            cnt = cnt + part_v[pl.ds(rec, 16)]
            s0 = s0 + part_v[pl.ds(rec + 16, 16)]
            s1 = s1 + part_v[pl.ds(rec + 32, 16)]
            return cnt, s0, s1

        cnt, s0, s1 = sums
        safe = jnp.maximum(cnt, jnp.float32(1.0))
        vd = jnp.where(cnt > jnp.float32(1.0), jnp.float32(1.0), jnp.float32(0.0)) * lbl_ok
        m0_t[pl.ds(img * 16, 16)] = s0 / safe
        m1_t[pl.ds(img * 16, 16)] = s1 / safe
        cf_t[pl.ds(img * 16, 16)] = vd / safe
        vd_t[pl.ds(img * 16, 16)] = vd

    acc_total = _zeros16()
    for img in range(NIMG):
        b = img % 2
        x0_v, x1_v, t_v = bufs[b]
        for cp in pending:
            cp.wait()
        if img + 1 < NIMG:
            pending = start(img + 1, 1 - b)
        m0_img = m0_t.at[pl.ds(img * 16, 16)]
        m1_img = m1_t.at[pl.ds(img * 16, 16)]
        cf_img = cf_t.at[pl.ds(img * 16, 16)]

        @plsc.parallel_loop(0, CHUNKS, 1, unroll=4, carry=_zeros16())
        def acc_img(i, acc):
            r, c = _rc(i)
            col = jnp.full((16,), c, I32) + lanes
            row = jnp.full((16,), r, I32)
            t16 = plsc.load_gather(t_v, [row, col])
            x0 = plsc.load_gather(x0_v, [row, col])
            x1 = plsc.load_gather(x1_v, [row, col])
            m0 = plsc.load_gather(m0_img, [t16])
            m1 = plsc.load_gather(m1_img, [t16])
            cf = plsc.load_gather(cf_img, [t16])
            dx = x0 - m0
            dy = x1 - m1
            s = dx * dx + dy * dy + jnp.float32(1e-8)
            d = s * _rsqrt(s)
            u = jnp.maximum(d - jnp.float32(0.5), jnp.float32(0.0))
            return acc + u * u * cf

        acc_total = acc_total + acc_img

    img_off = jnp.full((16,), lax.rem(wid, NIMG) * 16, I32)
    accp = _zeros16()
    for half in range(2):
        pi = pair_v[pl.ds(half * 16, 16)]
        pj = pair_v[pl.ds(32 + half * 16, 16)]
        mi0 = plsc.load_gather(m0_t, [img_off + pi])
        mj0 = plsc.load_gather(m0_t, [img_off + pj])
        mi1 = plsc.load_gather(m1_t, [img_off + pi])
        mj1 = plsc.load_gather(m1_t, [img_off + pj])
        vi = plsc.load_gather(vd_t, [img_off + pi])
        vj = plsc.load_gather(vd_t, [img_off + pj])
        dmx = mi0 - mj0
        dmy = mi1 - mj1
        sp = dmx * dmx + dmy * dmy + jnp.float32(1e-8)
        dp = sp * _rsqrt(sp)
        up = jnp.maximum(jnp.float32(1.0) - dp, jnp.float32(0.0))
        pm = vi * vj * jnp.where(pi < pj, jnp.float32(1.0), jnp.float32(0.0))
        accp = accp + up * up * pm

    vdrow = plsc.load_gather(vd_t, [img_off + lanes])
    n_v = jnp.full((16,), jnp.sum(vdrow), F32)
    ld_v = jnp.full((16,), jnp.sum(accp), F32)
    dist_v = ld_v / (n_v - jnp.float32(1.0) + jnp.float32(1e-8))
    wid_v = jnp.full((16,), wid, I32)
    take = (lanes == 0) & (n_v > jnp.float32(1.0)) & (wid_v < NIMG)
    dist_row = jnp.where(take, dist_v, _zeros16())

    row = (acc_total + dist_row) * jnp.float32(1.0 / NIMG)
    stage_v[pl.ds(0, 16)] = row
    pltpu.sync_copy(stage_v, out_hbm.at[pl.ds(wid * 16, 16)])


def _pair_table():
    pi = []
    pj = []
    for i in range(1, 9):
        for j in range(i + 1, 9):
            pi.append(i)
            pj.append(j)
    while len(pi) < 32:
        pi.append(0)
        pj.append(0)
    return jnp.asarray(pi + pj, dtype=jnp.int32)


def _pix_bufs():
    return [
        (
            pltpu.VMEM((ROWS_PW, W), F32),
            pltpu.VMEM((ROWS_PW, W), F32),
            pltpu.VMEM((ROWS_PW, W), I32),
        )
        for _ in range(2)
    ]


@jax.jit
def kernel(inputs, targets):
    x = inputs.reshape(NIMG * NCH * H, W)
    t = targets.reshape(NIMG * H, W).astype(jnp.int32)
    pairs = _pair_table()

    mesh = plsc.VectorSubcoreMesh(core_axis_name="c", subcore_axis_name="s")
    params = pltpu.CompilerParams(needs_layout_passes=False, use_tc_tiling_on_sc=True)

    phase_a = pl.kernel(
        _phase_a,
        out_type=jax.ShapeDtypeStruct((NW * REC,), F32),
        mesh=mesh,
        scratch_types=[
            _pix_bufs(),
            [pltpu.VMEM((NIMG * NBANK * NLBL * 16,), F32) for _ in range(3)],
            pltpu.VMEM((REC,), F32),
            [pltpu.SemaphoreType.DMA for _ in range(2)],
        ],
        compiler_params=params,
        name="instance_loss_segsums",
    )
    part = phase_a(x, t)

    phase_b = pl.kernel(
        _phase_b,
        out_type=jax.ShapeDtypeStruct((NW * 16,), F32),
        mesh=mesh,
        scratch_types=[
            pltpu.VMEM((NW * REC,), F32),
            _pix_bufs(),
            [pltpu.VMEM((NIMG * 16,), F32) for _ in range(4)],
            pltpu.VMEM((64,), I32),
            pltpu.VMEM((16,), F32),
            [pltpu.SemaphoreType.DMA for _ in range(2)],
        ],
        compiler_params=params,
        name="instance_loss_var_dist",
    )
    out = phase_b(x, t, part, pairs)
    return jnp.sum(out).reshape(1)
